# baseline (speedup 1.0000x reference)
import functools

import jax
import jax.numpy as jnp
from jax import lax
from jax.experimental import pallas as pl
from jax.experimental.pallas import tpu as pltpu
from jax.experimental.pallas import tpu_sc as plsc

N = 100000
C = 64
K = 27
EK = 50000
TOT = K * EK
NW = 32
PER_W = 43008
PAD = NW * PER_W
GCH = 384
GNB = GCH // 128
GNC = PER_W // GCH
SEG = 2048
NSEG = PER_W // SEG
RV = 20224
RP = RV // 2
P = 5
SPR = 10368
ZCH = 64
PPW = RP // 16
DUMMY = 1 << 28

_mesh = plsc.VectorSubcoreMesh(core_axis_name="c", subcore_axis_name="s")


@functools.partial(
    pl.kernel,
    mesh=_mesh,
    out_type=jax.ShapeDtypeStruct((PAD, 128), jnp.float32),
    scratch_types=[
        pltpu.VMEM((GCH,), jnp.int32),
        pltpu.VMEM((GCH,), jnp.int32),
        pltpu.VMEM((GCH, 128), jnp.float32),
        pltpu.VMEM((GCH, 128), jnp.float32),
        pltpu.SemaphoreType.DMA,
        pltpu.SemaphoreType.DMA,
        pltpu.SemaphoreType.DMA,
        pltpu.SemaphoreType.DMA,
    ],
)
def _sc_gather(x2_hbm, idx_hbm, out_hbm,
               idxbuf0, idxbuf1, rowbuf0, rowbuf1, sg0, sg1, sw0, sw1):
    c = lax.axis_index("c")
    s = lax.axis_index("s")
    wid = c * 16 + s
    base = wid * PER_W

    def body(j, carry):
        off0 = base + (2 * j) * GCH
        off1 = off0 + GCH
        pltpu.sync_copy(idx_hbm.at[pl.ds(off0, GCH)], idxbuf0)

        @pl.when(j > 0)
        def _():
            pltpu.make_async_copy(
                rowbuf0, out_hbm.at[pl.ds(0, GCH), :], sw0).wait()

        g0 = [pltpu.async_copy(
            x2_hbm.at[idxbuf0.at[pl.ds(k * 128, 128)]],
            rowbuf0.at[pl.ds(k * 128, 128), :], sg0) for k in range(GNB)]
        pltpu.sync_copy(idx_hbm.at[pl.ds(off1, GCH)], idxbuf1)

        @pl.when(j > 0)
        def _():
            pltpu.make_async_copy(
                rowbuf1, out_hbm.at[pl.ds(0, GCH), :], sw1).wait()

        g1 = [pltpu.async_copy(
            x2_hbm.at[idxbuf1.at[pl.ds(k * 128, 128)]],
            rowbuf1.at[pl.ds(k * 128, 128), :], sg1) for k in range(GNB)]
        for h in g0:
            h.wait()
        pltpu.async_copy(rowbuf0, out_hbm.at[pl.ds(off0, GCH), :], sw0)
        for h in g1:
            h.wait()
        pltpu.async_copy(rowbuf1, out_hbm.at[pl.ds(off1, GCH), :], sw1)
        return carry

    lax.fori_loop(0, GNC // 2, body, 0)
    pltpu.make_async_copy(rowbuf0, out_hbm.at[pl.ds(0, GCH), :], sw0).wait()
    pltpu.make_async_copy(rowbuf1, out_hbm.at[pl.ds(0, GCH), :], sw1).wait()


@functools.partial(
    pl.kernel,
    mesh=_mesh,
    out_type=jax.ShapeDtypeStruct((2, P * RP, 128), jnp.float32),
    scratch_types=[
        pltpu.VMEM((2048,), jnp.int32),
        pltpu.VMEM((2048,), jnp.int32),
        pltpu.VMEM((1, 128), jnp.int32),
        pltpu.VMEM((1, 128), jnp.int32),
        pltpu.VMEM((128, 128), jnp.float32),
        pltpu.VMEM((128, 128), jnp.float32),
        pltpu.VMEM((ZCH, 128), jnp.float32),
        pltpu.VMEM_SHARED((SPR, 128), jnp.float32),
        pltpu.SemaphoreType.DMA,
        pltpu.SemaphoreType.DMA,
        pltpu.SemaphoreType.DMA,
        pltpu.SemaphoreType.DMA,
        pltpu.SemaphoreType.DMA,
        pltpu.SemaphoreType.DMA,
    ],
)
def _sc_scatter(oidx_hbm, msg_hbm, zeros_hbm, part_hbm,
                oidxA, oidxB, clocA, clocB, stageA, stageB,
                zerobuf, spmem, sga, sgb, ssa, ssb, soa, sob):
    c = lax.axis_index("c")
    s = lax.axis_index("s")
    wid = c * 16 + s
    base = wid * PER_W
    pltpu.sync_copy(zeros_hbm, zerobuf)

    def adj(buf, boff, cl, lo):
        for i in range(8):
            v = buf[pl.ds(boff + i * 16, 16)]
            loc = v - lo
            ok = (loc >= 0) & (loc < RV)
            pr = lax.shift_right_logical(loc, 1)
            dummy = RP + (lax.shift_right_logical(v, 1) & 255)
            cl[0, pl.ds(i * 16, 16)] = jnp.where(ok, pr, dummy)

    def phase(p, carry):
        plsc.subcore_barrier()
        for z in range(10):
            pltpu.sync_copy(
                zerobuf, spmem.at[pl.ds((s * 10 + z) * ZCH, ZCH), :])
        plsc.subcore_barrier()
        lo = p * RV

        pltpu.async_copy(oidx_hbm.at[pl.ds(base, 2048)], oidxA, soa)
        pltpu.async_copy(oidx_hbm.at[pl.ds(base + 2048, 2048)], oidxB, sob)

        def process_seg(u, sg, buf, first):
            for pi in range(8):
                off = base + sg * 2048 + pi * 256
                if first and pi == 0:
                    @pl.when(u > 0)
                    def _():
                        pltpu.make_async_copy(
                            stageA, spmem.at[clocA.at[0]], ssa).wait()
                        pltpu.make_async_copy(
                            stageB, spmem.at[clocB.at[0]], ssb).wait()
                else:
                    pltpu.make_async_copy(
                        stageA, spmem.at[clocA.at[0]], ssa).wait()
                    pltpu.make_async_copy(
                        stageB, spmem.at[clocB.at[0]], ssb).wait()
                ga = pltpu.async_copy(
                    msg_hbm.at[pl.ds(off, 128), :], stageA, sga)
                gb = pltpu.async_copy(
                    msg_hbm.at[pl.ds(off + 128, 128), :], stageB, sgb)
                adj(buf, pi * 256, clocA, lo)
                adj(buf, pi * 256 + 128, clocB, lo)
                ga.wait()
                pltpu.async_copy(
                    stageA, spmem.at[clocA.at[0]], ssa, add=True)
                gb.wait()
                pltpu.async_copy(
                    stageB, spmem.at[clocB.at[0]], ssb, add=True)

        def dseg(u, carry2):
            sg0 = u * 2
            pltpu.make_async_copy(
                oidx_hbm.at[pl.ds(0, 2048)], oidxA, soa).wait()
            process_seg(u, sg0, oidxA, True)

            pltpu.async_copy(
                oidx_hbm.at[pl.ds(base + (sg0 + 2) * 2048, 2048)],
                oidxA, soa)
            pltpu.make_async_copy(
                oidx_hbm.at[pl.ds(0, 2048)], oidxB, sob).wait()
            process_seg(u, sg0 + 1, oidxB, False)

            @pl.when(u < 9)
            def _():
                pltpu.async_copy(
                    oidx_hbm.at[pl.ds(base + (sg0 + 3) * 2048, 2048)],
                    oidxB, sob)
            return carry2

        lax.fori_loop(0, 10, dseg, 0)
        pltpu.make_async_copy(oidx_hbm.at[pl.ds(0, 2048)], oidxA, soa).wait()
        process_seg(jnp.int32(1), 20, oidxA, False)
        pltpu.make_async_copy(stageA, spmem.at[clocA.at[0]], ssa).wait()
        pltpu.make_async_copy(stageB, spmem.at[clocB.at[0]], ssb).wait()
        plsc.subcore_barrier()

        pltpu.sync_copy(
            spmem.at[pl.ds(s * PPW, PPW), :],
            part_hbm.at[c, pl.ds(p * RP + s * PPW, PPW), :],
        )
        return carry

    lax.fori_loop(0, P, phase, 0)


def _mm_body(a_ref, w_ref, o_ref):
    o_ref[...] = lax.dot_general(
        a_ref[...].astype(jnp.bfloat16), w_ref[0],
        (((1,), (0,)), ((), ())),
        preferred_element_type=jnp.float32,
    )


_mm = pl.pallas_call(
    _mm_body,
    grid=(K, 25),
    in_specs=[
        pl.BlockSpec((2000, 128), lambda k, i: (k * 25 + i, 0)),
        pl.BlockSpec((1, 128, 128), lambda k, i: (k, 0, 0)),
    ],
    out_specs=pl.BlockSpec((2000, 128), lambda k, i: (k * 25 + i, 0)),
    out_shape=jax.ShapeDtypeStruct((PAD, 128), jnp.float32),

)


def _add_body(a_ref, b_ref, o_ref):
    o_ref[...] = a_ref[...] + b_ref[...]


_addk = pl.pallas_call(
    _add_body,
    grid=(40,),
    in_specs=[
        pl.BlockSpec((1264, 128), lambda i: (i, 0)),
        pl.BlockSpec((1264, 128), lambda i: (i, 0)),
    ],
    out_specs=pl.BlockSpec((1264, 128), lambda i: (i, 0)),
    out_shape=jax.ShapeDtypeStruct((P * RP, 128), jnp.float32),
)


def kernel(x, in_idx, out_idx, kernel):
    in_flat = in_idx.reshape(-1).astype(jnp.int32)
    out_flat = out_idx.reshape(-1).astype(jnp.int32)
    x2 = jnp.concatenate(
        [jnp.pad(x, ((0, 0), (0, 64))), jnp.pad(x, ((0, 0), (64, 0)))])
    gidx = in_flat + N * (out_flat & 1)
    ar = jnp.arange(PAD - TOT, dtype=jnp.int32)
    in_pad = jnp.concatenate([gidx, ar % N])
    out_pad = jnp.concatenate([out_flat, DUMMY + 2 * (ar % 128)])
    wb = kernel.astype(jnp.bfloat16)
    w2 = jnp.zeros((K, 128, 128), jnp.bfloat16)
    w2 = w2.at[:, :C, :C].set(wb).at[:, C:, C:].set(wb)
    zeros = jnp.zeros((ZCH, 128), jnp.float32)

    gathered = _sc_gather(x2, in_pad)
    msg2 = _mm(gathered, w2)
    parts = _sc_scatter(out_pad, msg2, zeros)
    res = _addk(parts[0], parts[1])
    return res[:N // 2].reshape(N, C)


# speedup vs baseline: 1.0077x; 1.0077x over previous
import functools

import jax
import jax.numpy as jnp
from jax import lax
from jax.experimental import pallas as pl
from jax.experimental.pallas import tpu as pltpu
from jax.experimental.pallas import tpu_sc as plsc

N = 100000
C = 64
K = 27
EK = 50000
TOT = K * EK
NW = 32
PER_W = 43008
PAD = NW * PER_W
GCH = 384
GNB = GCH // 128
GNC = PER_W // GCH
SEG = 2048
NSEG = PER_W // SEG
RV = 20224
RP = RV // 2
P = 5
SPR = 10240
ZCH = 64
PPW = RP // 16
DUMMY = 1 << 28

_mesh = plsc.VectorSubcoreMesh(core_axis_name="c", subcore_axis_name="s")


@functools.partial(
    pl.kernel,
    mesh=_mesh,
    out_type=jax.ShapeDtypeStruct((PAD, 128), jnp.float32),
    scratch_types=[
        pltpu.VMEM((GCH,), jnp.int32),
        pltpu.VMEM((GCH,), jnp.int32),
        pltpu.VMEM((GCH, 128), jnp.float32),
        pltpu.VMEM((GCH, 128), jnp.float32),
        pltpu.SemaphoreType.DMA,
        pltpu.SemaphoreType.DMA,
        pltpu.SemaphoreType.DMA,
        pltpu.SemaphoreType.DMA,
    ],
)
def _sc_gather(x2_hbm, idx_hbm, out_hbm,
               idxbuf0, idxbuf1, rowbuf0, rowbuf1, sg0, sg1, sw0, sw1):
    c = lax.axis_index("c")
    s = lax.axis_index("s")
    wid = c * 16 + s
    base = wid * PER_W

    def body(j, carry):
        off0 = base + (2 * j) * GCH
        off1 = off0 + GCH
        pltpu.sync_copy(idx_hbm.at[pl.ds(off0, GCH)], idxbuf0)

        @pl.when(j > 0)
        def _():
            pltpu.make_async_copy(
                rowbuf0, out_hbm.at[pl.ds(0, GCH), :], sw0).wait()

        g0 = [pltpu.async_copy(
            x2_hbm.at[idxbuf0.at[pl.ds(k * 128, 128)]],
            rowbuf0.at[pl.ds(k * 128, 128), :], sg0) for k in range(GNB)]
        pltpu.sync_copy(idx_hbm.at[pl.ds(off1, GCH)], idxbuf1)

        @pl.when(j > 0)
        def _():
            pltpu.make_async_copy(
                rowbuf1, out_hbm.at[pl.ds(0, GCH), :], sw1).wait()

        g1 = [pltpu.async_copy(
            x2_hbm.at[idxbuf1.at[pl.ds(k * 128, 128)]],
            rowbuf1.at[pl.ds(k * 128, 128), :], sg1) for k in range(GNB)]
        for h in g0:
            h.wait()
        pltpu.async_copy(rowbuf0, out_hbm.at[pl.ds(off0, GCH), :], sw0)
        for h in g1:
            h.wait()
        pltpu.async_copy(rowbuf1, out_hbm.at[pl.ds(off1, GCH), :], sw1)
        return carry

    lax.fori_loop(0, GNC // 2, body, 0)
    pltpu.make_async_copy(rowbuf0, out_hbm.at[pl.ds(0, GCH), :], sw0).wait()
    pltpu.make_async_copy(rowbuf1, out_hbm.at[pl.ds(0, GCH), :], sw1).wait()


@functools.partial(
    pl.kernel,
    mesh=_mesh,
    out_type=jax.ShapeDtypeStruct((2, P * RP, 128), jnp.float32),
    scratch_types=[
        pltpu.VMEM((256,), jnp.int32),
        pltpu.VMEM((1, 128), jnp.int32),
        pltpu.VMEM((1, 128), jnp.int32),
        pltpu.VMEM((128, 128), jnp.float32),
        pltpu.VMEM((128, 128), jnp.float32),
        pltpu.VMEM((ZCH, 128), jnp.float32),
        pltpu.VMEM_SHARED((SPR, 128), jnp.float32),
        pltpu.SemaphoreType.DMA,
        pltpu.SemaphoreType.DMA,
        pltpu.SemaphoreType.DMA,
        pltpu.SemaphoreType.DMA,
    ],
)
def _sc_scatter(oidx_hbm, msg_hbm, zeros_hbm, part_hbm,
                oidxbuf, clocA, clocB, stageA, stageB,
                zerobuf, spmem, sga, sgb, ssa, ssb):
    c = lax.axis_index("c")
    s = lax.axis_index("s")
    wid = c * 16 + s
    base = wid * PER_W
    pltpu.sync_copy(zeros_hbm, zerobuf)

    def phase(p, carry):
        plsc.subcore_barrier()
        for z in range(10):
            pltpu.sync_copy(
                zerobuf, spmem.at[pl.ds((s * 10 + z) * ZCH, ZCH), :])
        plsc.subcore_barrier()
        lo = p * RV

        def pair(j, carry2):
            off = base + j * 256
            pltpu.sync_copy(oidx_hbm.at[pl.ds(off, 256)], oidxbuf)

            @pl.when(j > 0)
            def _():
                pltpu.make_async_copy(
                    stageA, spmem.at[clocA.at[0]], ssa).wait()

            ga = pltpu.async_copy(
                msg_hbm.at[pl.ds(off, 128), :], stageA, sga)
            for i in range(8):
                v = oidxbuf[pl.ds(i * 16, 16)]
                loc = v - lo
                ok = (loc >= 0) & (loc < RV)
                pair_r = lax.shift_right_logical(loc, 1)
                dummy = RP + (lax.shift_right_logical(v, 1) & 127)
                clocA[0, pl.ds(i * 16, 16)] = jnp.where(ok, pair_r, dummy)

            @pl.when(j > 0)
            def _():
                pltpu.make_async_copy(
                    stageB, spmem.at[clocB.at[0]], ssb).wait()

            gb = pltpu.async_copy(
                msg_hbm.at[pl.ds(off + 128, 128), :], stageB, sgb)
            for i in range(8):
                v = oidxbuf[pl.ds(128 + i * 16, 16)]
                loc = v - lo
                ok = (loc >= 0) & (loc < RV)
                pair_r = lax.shift_right_logical(loc, 1)
                dummy = RP + (lax.shift_right_logical(v, 1) & 127)
                clocB[0, pl.ds(i * 16, 16)] = jnp.where(ok, pair_r, dummy)
            ga.wait()
            pltpu.async_copy(stageA, spmem.at[clocA.at[0]], ssa, add=True)
            gb.wait()
            pltpu.async_copy(stageB, spmem.at[clocB.at[0]], ssb, add=True)
            return carry2

        lax.fori_loop(0, PER_W // 256, pair, 0)
        pltpu.make_async_copy(stageA, spmem.at[clocA.at[0]], ssa).wait()
        pltpu.make_async_copy(stageB, spmem.at[clocB.at[0]], ssb).wait()
        plsc.subcore_barrier()

        pltpu.sync_copy(
            spmem.at[pl.ds(s * PPW, PPW), :],
            part_hbm.at[c, pl.ds(p * RP + s * PPW, PPW), :],
        )
        return carry

    lax.fori_loop(0, P, phase, 0)


def _mm_body(a_ref, w_ref, o_ref):
    o_ref[...] = lax.dot_general(
        a_ref[...].astype(jnp.bfloat16), w_ref[0],
        (((1,), (0,)), ((), ())),
        preferred_element_type=jnp.float32,
    )


_mm = pl.pallas_call(
    _mm_body,
    grid=(K, 25),
    in_specs=[
        pl.BlockSpec((2000, 128), lambda k, i: (k * 25 + i, 0)),
        pl.BlockSpec((1, 128, 128), lambda k, i: (k, 0, 0)),
    ],
    out_specs=pl.BlockSpec((2000, 128), lambda k, i: (k * 25 + i, 0)),
    out_shape=jax.ShapeDtypeStruct((PAD, 128), jnp.float32),

)


def _add_body(a_ref, b_ref, o_ref):
    o_ref[...] = a_ref[...] + b_ref[...]


_addk = pl.pallas_call(
    _add_body,
    grid=(40,),
    in_specs=[
        pl.BlockSpec((1264, 128), lambda i: (i, 0)),
        pl.BlockSpec((1264, 128), lambda i: (i, 0)),
    ],
    out_specs=pl.BlockSpec((1264, 128), lambda i: (i, 0)),
    out_shape=jax.ShapeDtypeStruct((P * RP, 128), jnp.float32),
)


def kernel(x, in_idx, out_idx, kernel):
    in_flat = in_idx.reshape(-1).astype(jnp.int32)
    out_flat = out_idx.reshape(-1).astype(jnp.int32)
    x2 = jnp.concatenate(
        [jnp.pad(x, ((0, 0), (0, 64))), jnp.pad(x, ((0, 0), (64, 0)))])
    gidx = in_flat + N * (out_flat & 1)
    ar = jnp.arange(PAD - TOT, dtype=jnp.int32)
    in_pad = jnp.concatenate([gidx, ar % N])
    out_pad = jnp.concatenate([out_flat, DUMMY + 2 * (ar % 128)])
    wb = kernel.astype(jnp.bfloat16)
    w2 = jnp.zeros((K, 128, 128), jnp.bfloat16)
    w2 = w2.at[:, :C, :C].set(wb).at[:, C:, C:].set(wb)
    zeros = jnp.zeros((ZCH, 128), jnp.float32)

    gathered = _sc_gather(x2, in_pad)
    msg2 = _mm(gathered, w2)
    parts = _sc_scatter(out_pad, msg2, zeros)
    res = _addk(parts[0], parts[1])
    return res[:N // 2].reshape(N, C)
